# Initial kernel scaffold; baseline (speedup 1.0000x reference)
import jax
import jax.numpy as jnp
from jax.experimental import pallas as pl


def kernel(scores):
    raise NotImplementedError("write your pallas kernel here")


# speedup vs baseline: 9.8536x; 9.8536x over previous
import jax
import jax.numpy as jnp
from jax.experimental import pallas as pl

LAMBDA = 1.1
ROWS_PER_BLOCK = 8
TILE = 2048
MAX_MICHELOT_ITERS = 14


def _body(s_ref, u_ref, out_ref, ent_ref):
    K = s_ref.shape[1]
    n_full = K // TILE
    tail = K - n_full * TILE
    kf = jnp.float32(K)

    def tile_slice(i):
        return pl.ds(i * TILE, TILE)

    def p1(i, carry):
        m_a, m_s = carry
        s = s_ref[:, tile_slice(i)]
        u = u_ref[:, tile_slice(i)]
        a = s - jnp.log(-jnp.log(u))
        out_ref[:, tile_slice(i)] = a
        m_a = jnp.maximum(m_a, jnp.max(a, axis=1, keepdims=True))
        m_s = jnp.maximum(m_s, jnp.max(s, axis=1, keepdims=True))
        return m_a, m_s

    neg_inf = jnp.full((ROWS_PER_BLOCK, 1), -jnp.inf, jnp.float32)
    m_a, m_s = jax.lax.fori_loop(0, n_full, p1, (neg_inf, neg_inf))
    if tail:
        ts = pl.ds(n_full * TILE, tail)
        s = s_ref[:, ts]
        a = s - jnp.log(-jnp.log(u_ref[:, ts]))
        out_ref[:, ts] = a
        m_a = jnp.maximum(m_a, jnp.max(a, axis=1, keepdims=True))
        m_s = jnp.maximum(m_s, jnp.max(s, axis=1, keepdims=True))

    def p2(i, carry):
        d_a, d_s, dot = carry
        a = out_ref[:, tile_slice(i)]
        w = jnp.exp(a - m_a)
        out_ref[:, tile_slice(i)] = w
        s = s_ref[:, tile_slice(i)]
        es = jnp.exp(s - m_s)
        d_a = d_a + jnp.sum(w, axis=1, keepdims=True)
        d_s = d_s + jnp.sum(es, axis=1, keepdims=True)
        dot = dot + jnp.sum(es * s, axis=1, keepdims=True)
        return d_a, d_s, dot

    zero = jnp.zeros((ROWS_PER_BLOCK, 1), jnp.float32)
    d_a, d_s, dot = jax.lax.fori_loop(0, n_full, p2, (zero, zero, zero))
    if tail:
        ts = pl.ds(n_full * TILE, tail)
        a = out_ref[:, ts]
        w = jnp.exp(a - m_a)
        out_ref[:, ts] = w
        s = s_ref[:, ts]
        es = jnp.exp(s - m_s)
        d_a = d_a + jnp.sum(w, axis=1, keepdims=True)
        d_s = d_s + jnp.sum(es, axis=1, keepdims=True)
        dot = dot + jnp.sum(es * s, axis=1, keepdims=True)

    ent_ref[...] = m_s + jnp.log(d_s) - dot / d_s

    target = d_a / LAMBDA

    def sweep(t):
        def acc(i, carry):
            S, N = carry
            w = out_ref[:, tile_slice(i)]
            mask = w >= t
            S = S + jnp.sum(jnp.where(mask, w, 0.0), axis=1, keepdims=True)
            N = N + jnp.sum(jnp.where(mask, 1.0, 0.0), axis=1, keepdims=True)
            return S, N

        S, N = jax.lax.fori_loop(0, n_full, acc, (zero, zero))
        if tail:
            w = out_ref[:, pl.ds(n_full * TILE, tail)]
            mask = w >= t
            S = S + jnp.sum(jnp.where(mask, w, 0.0), axis=1, keepdims=True)
            N = N + jnp.sum(jnp.where(mask, 1.0, 0.0), axis=1, keepdims=True)
        return (S - target) / N

    def cond(carry):
        it, _, done = carry
        return jnp.logical_and(it < MAX_MICHELOT_ITERS, jnp.logical_not(done))

    def step(carry):
        it, t, _ = carry
        t_new = sweep(t)
        return it + 1, t_new, jnp.all(t_new == t)

    t0 = (d_a - target) / kf
    _, t, _ = jax.lax.while_loop(cond, step, (jnp.int32(0), t0, jnp.bool_(False)))

    scale = LAMBDA / d_a

    def p4(i, _):
        w = out_ref[:, tile_slice(i)]
        out_ref[:, tile_slice(i)] = jnp.maximum(w - t, 0.0) * scale
        return 0

    jax.lax.fori_loop(0, n_full, p4, 0)
    if tail:
        ts = pl.ds(n_full * TILE, tail)
        w = out_ref[:, ts]
        out_ref[:, ts] = jnp.maximum(w - t, 0.0) * scale


def _run(scores, u):
    R, K = scores.shape
    grid = (R // ROWS_PER_BLOCK,)
    sample, ent = pl.pallas_call(
        _body,
        grid=grid,
        in_specs=[
            pl.BlockSpec((ROWS_PER_BLOCK, K), lambda i: (i, 0)),
            pl.BlockSpec((ROWS_PER_BLOCK, K), lambda i: (i, 0)),
        ],
        out_specs=[
            pl.BlockSpec((ROWS_PER_BLOCK, K), lambda i: (i, 0)),
            pl.BlockSpec((ROWS_PER_BLOCK, 1), lambda i: (i, 0)),
        ],
        out_shape=[
            jax.ShapeDtypeStruct((R, K), jnp.float32),
            jax.ShapeDtypeStruct((R, 1), jnp.float32),
        ],
    )(scores, u)
    return sample, ent


def kernel(scores):
    u = jax.random.uniform(
        jax.random.key(42), scores.shape, scores.dtype, minval=1e-10, maxval=1.0
    )
    sample, ent = _run(scores, u)
    return sample, scores, ent.reshape(scores.shape[0])
